# baseline (speedup 1.0000x reference)
import functools

import jax
import jax.numpy as jnp
from jax import lax
from jax.experimental import pallas as pl
from jax.experimental.pallas import tpu as pltpu
from jax.experimental.pallas import tpu_sc as plsc

_N = 1000
_D = 4096
_CHUNK_ROWS = 8
_NCHUNKS = _N // _CHUNK_ROWS
_NW = 32
_NS = 16

_mesh = plsc.VectorSubcoreMesh(core_axis_name="c", subcore_axis_name="s")


@functools.partial(
    pl.kernel,
    mesh=_mesh,
    out_type=jax.ShapeDtypeStruct((_N, _D), jnp.float32),
    scratch_types=[
        pltpu.VMEM_SHARED((_NS, 3, _CHUNK_ROWS, _D), jnp.float32),
        pltpu.SemaphoreType.DMA,
        pltpu.SemaphoreType.DMA,
        pltpu.SemaphoreType.DMA,
        pltpu.SemaphoreType.DMA,
        pltpu.SemaphoreType.DMA,
        pltpu.SemaphoreType.DMA,
    ],
)
def _copy_kernel(src_hbm, out_hbm, shared, si0, si1, si2, so0, so1, so2):
    cid = lax.axis_index("c")
    sid = lax.axis_index("s")
    wid = sid * 2 + cid
    sins = (si0, si1, si2)
    souts = (so0, so1, so2)

    def start_in(i):
        r = (wid + _NW * i) * _CHUNK_ROWS
        return pltpu.async_copy(
            src_hbm.at[pl.ds(r, _CHUNK_ROWS), :],
            shared.at[sid, i % 3], sins[i % 3])

    def start_out(i):
        r = (wid + _NW * i) * _CHUNK_ROWS
        return pltpu.async_copy(
            shared.at[sid, i % 3],
            out_hbm.at[pl.ds(r, _CHUNK_ROWS), :], souts[i % 3])

    h_in0 = start_in(0)
    h_in1 = start_in(1)
    h_in2 = start_in(2)
    h_in0.wait()
    h_out0 = start_out(0)
    h_in1.wait()
    h_out1 = start_out(1)
    h_in2.wait()
    h_out2 = start_out(2)
    h_out0.wait()

    @pl.when(wid + _NW * 3 < _NCHUNKS)
    def _():
        h_in3 = start_in(3)
        h_in3.wait()
        h_out3 = start_out(3)
        h_out3.wait()

    h_out1.wait()
    h_out2.wait()


def kernel(special_embeddings_weight):
    return _copy_kernel(special_embeddings_weight)
